# baseline (speedup 1.0000x reference)
import functools

import numpy as np
import jax
import jax.numpy as jnp
from jax.experimental import pallas as pl
from jax.experimental.pallas import tpu as pltpu

_D = 512
_H = 8
_E = 64
_FF = 2048
_TF = 4
_PRED_LEN = 1024
_PREC = jax.lax.Precision.DEFAULT


def _u_of(L):
    return min(int(5 * np.ceil(np.log(L))), L)


def _tf2x32(k0, k1, c0, c1):
    u32 = np.uint32
    rotations = [[13, 15, 26, 6], [17, 29, 16, 24]]
    ks = [u32(k0), u32(k1), u32(k0) ^ u32(k1) ^ u32(0x1BD11BDA)]
    x = [(c0 + ks[0]).astype(u32), (c1 + ks[1]).astype(u32)]
    for i in range(5):
        for r in rotations[i % 2]:
            x[0] = (x[0] + x[1]).astype(u32)
            x[1] = (((x[1] << u32(r)) | (x[1] >> u32(32 - r))).astype(u32)
                    ^ x[0])
        x[0] = (x[0] + ks[(i + 1) % 3]).astype(u32)
        x[1] = (x[1] + ks[(i + 2) % 3] + u32(i + 1)).astype(u32)
    return x[0], x[1]


def _np_randint(key, shape, maxval):
    o0, o1 = _tf2x32(key[0], key[1], np.zeros(2, np.uint32),
                     np.arange(2, dtype=np.uint32))
    k1 = np.array([o0[0], o1[0]], np.uint32)
    k2 = np.array([o0[1], o1[1]], np.uint32)

    def bits(k, n):
        b0, b1 = _tf2x32(k[0], k[1], np.zeros(n, np.uint32),
                         np.arange(n, dtype=np.uint32))
        return b0 ^ b1

    n = int(np.prod(shape))
    higher = bits(k1, n).reshape(shape)
    lower = bits(k2, n).reshape(shape)
    span = np.uint32(maxval)
    mult = ((np.uint32(65536) % span) * (np.uint32(65536) % span)) % span
    return (((higher % span) * mult + (lower % span)) % span).astype(np.int32)


@functools.lru_cache(maxsize=None)
def _sample_counts(fold, L_Q, L_K):
    key = _tf2x32(np.uint32(0), np.uint32(1234),
                  np.zeros(1, np.uint32), np.full(1, fold, np.uint32))
    key = np.array([key[0][0], key[1][0]], np.uint32)
    idx = _np_randint(key, (L_Q, _u_of(L_K)), L_K)
    C = np.zeros((L_Q, L_K), np.int8)
    np.add.at(C, (np.arange(L_Q)[:, None], idx), 1)
    return C


@functools.lru_cache(maxsize=None)
def _pos_table(L, d=_D):
    pos = np.arange(L, dtype=np.float32)[:, None]
    div = np.exp(np.arange(0, d, 2, dtype=np.float32) * (-np.log(10000.0) / d))
    pe = np.zeros((L, d), np.float32)
    pe[:, 0::2] = np.sin(pos * div)
    pe[:, 1::2] = np.cos(pos * div)
    return pe


def _mm(x, w, *, bias=None, add=None, add_period=None, res=None, ln=None,
        act=None, scale=1.0, blk=256):
    M, K = x.shape
    N = w.shape[1]
    assert M % blk == 0, (M, blk)
    inputs = [x, w]
    specs = [pl.BlockSpec((blk, K), lambda i: (i, 0)),
             pl.BlockSpec((K, N), lambda i: (0, 0))]
    if bias is not None:
        inputs.append(bias.reshape(1, N))
        specs.append(pl.BlockSpec((1, N), lambda i: (0, 0)))
    if add is not None:
        nb = add.shape[0] // blk
        inputs.append(add)
        specs.append(pl.BlockSpec((blk, N), lambda i, _nb=nb: (i % _nb, 0)))
    if res is not None:
        inputs.append(res)
        specs.append(pl.BlockSpec((blk, N), lambda i: (i, 0)))
    if ln is not None:
        g, b = ln
        inputs += [g.reshape(1, N), b.reshape(1, N)]
        specs += [pl.BlockSpec((1, N), lambda i: (0, 0)),
                  pl.BlockSpec((1, N), lambda i: (0, 0))]

    def body(*refs):
        x_ref, w_ref, *rest = refs[:-1]
        o_ref = refs[-1]
        y = jax.lax.dot_general(x_ref[...], w_ref[...], (((1,), (0,)), ((), ())),
                                precision=_PREC,
                                preferred_element_type=jnp.float32)
        it = iter(rest)
        if bias is not None:
            y = y + next(it)[...]
        if add is not None:
            y = y + next(it)[...]
        if scale != 1.0:
            y = y * np.float32(scale)
        if act == "gelu":
            y = 0.5 * y * (1.0 + jax.lax.erf(y * np.float32(1.0 / np.sqrt(2.0))))
        elif act == "elu":
            y = jnp.where(y > 0, y, jnp.exp(jnp.minimum(y, 0.0)) - 1.0)
        if res is not None:
            y = y + next(it)[...]
        if ln is not None:
            g_ref = next(it)
            b_ref = next(it)
            m = jnp.mean(y, axis=-1, keepdims=True)
            d = y - m
            v = jnp.mean(d * d, axis=-1, keepdims=True)
            y = d * jax.lax.rsqrt(v + 1e-5) * g_ref[...] + b_ref[...]
        o_ref[...] = y

    return pl.pallas_call(
        body,
        grid=(M // blk,),
        in_specs=specs,
        out_specs=pl.BlockSpec((blk, N), lambda i: (i, 0)),
        out_shape=jax.ShapeDtypeStruct((M, N), jnp.float32),
    )(*inputs)


def _layernorm(x, g, b, blk=256):
    M, N = x.shape

    def body(x_ref, g_ref, b_ref, o_ref):
        y = x_ref[...]
        m = jnp.mean(y, axis=-1, keepdims=True)
        d = y - m
        v = jnp.mean(d * d, axis=-1, keepdims=True)
        o_ref[...] = d * jax.lax.rsqrt(v + 1e-5) * g_ref[...] + b_ref[...]

    return pl.pallas_call(
        body,
        grid=(M // blk,),
        in_specs=[pl.BlockSpec((blk, N), lambda i: (i, 0)),
                  pl.BlockSpec((1, N), lambda i: (0, 0)),
                  pl.BlockSpec((1, N), lambda i: (0, 0))],
        out_specs=pl.BlockSpec((blk, N), lambda i: (i, 0)),
        out_shape=jax.ShapeDtypeStruct((M, N), jnp.float32),
    )(x, g.reshape(1, N), b.reshape(1, N))


def _topk_queries(Qh, Kh, C, u):
    G, L_Q, E = Qh.shape
    L_K = Kh.shape[1]
    QB = min(1024, L_Q)
    nj = L_Q // QB
    rows = QB // 128

    def body(q_ref, k_ref, c_ref, o_ref, m_scr):
        j = pl.program_id(1)
        qk = jax.lax.dot_general(q_ref[0], k_ref[0], (((1,), (1,)), ((), ())),
                                 precision=_PREC,
                                 preferred_element_type=jnp.float32)
        c = c_ref[...].astype(jnp.float32)
        mx = jnp.max(jnp.where(c > 0, qk, np.float32(-1e30)), axis=1)
        sm = jnp.sum(qk * c, axis=1) * np.float32(1.0 / L_K)
        m_scr[pl.ds(j * rows, rows), :] = (mx - sm).reshape(rows, 128)

        @pl.when(j == nj - 1)
        def _():
            Mv = m_scr[...]
            gio = (jax.lax.broadcasted_iota(jnp.int32, Mv.shape, 0) * 128
                   + jax.lax.broadcasted_iota(jnp.int32, Mv.shape, 1))
            lane = jax.lax.broadcasted_iota(jnp.int32, (1, 128), 1)
            out = jnp.full((1, 128), -1, jnp.int32)
            for t in range(u):
                mv = jnp.max(Mv)
                gi = jnp.min(jnp.where(Mv == mv, gio, jnp.int32(2**30)))
                out = jnp.where(lane == t, gi, out)
                Mv = jnp.where(gio == gi, np.float32(-np.inf), Mv)
            o_ref[0] = out

    idx = pl.pallas_call(
        body,
        grid=(G, nj),
        in_specs=[pl.BlockSpec((1, QB, E), lambda i, j: (i, j, 0)),
                  pl.BlockSpec((1, L_K, E), lambda i, j: (i, 0, 0)),
                  pl.BlockSpec((QB, L_K), lambda i, j: (j, 0))],
        out_specs=pl.BlockSpec((1, 1, 128), lambda i, j: (i, 0, 0)),
        out_shape=jax.ShapeDtypeStruct((G, 1, 128), jnp.int32),
        scratch_shapes=[pltpu.VMEM((L_Q // 128, 128), jnp.float32)],
    )(Qh, Kh, C)
    return idx


def _attn_apply(idx, Qh, Kh, Vh, u, causal):
    G, L_Q, E = Qh.shape
    L_K = Kh.shape[1]
    UP = 64
    CH = 512

    def body(idx_ref, q_ref, k_ref, v_ref, o_ref):
        V = v_ref[0]
        idp = idx_ref[0][:, :UP]
        rio = jax.lax.broadcasted_iota(jnp.int32, (L_Q, UP), 0)
        ot = jnp.where(rio == idp, np.float32(1.0), np.float32(0.0))
        qr = jax.lax.dot_general(ot, q_ref[0], (((0,), (0,)), ((), ())),
                                 precision=_PREC,
                                 preferred_element_type=jnp.float32)
        scores = jax.lax.dot_general(qr, k_ref[0], (((1,), (1,)), ((), ())),
                                     precision=_PREC,
                                     preferred_element_type=jnp.float32)
        scores = scores * np.float32(1.0 / np.sqrt(E))
        if causal:
            pcol = jnp.transpose(idp, (1, 0))
            kio = jax.lax.broadcasted_iota(jnp.int32, (UP, L_K), 1)
            scores = jnp.where(kio > pcol, np.float32(-1e30), scores)
        smax = jnp.max(scores, axis=-1, keepdims=True)
        ex = jnp.exp(scores - smax)
        attn = ex / jnp.sum(ex, axis=-1, keepdims=True)
        upd = jax.lax.dot_general(attn, V, (((1,), (0,)), ((), ())),
                                  precision=_PREC,
                                  preferred_element_type=jnp.float32)
        scat = jax.lax.dot_general(ot, upd, (((1,), (0,)), ((), ())),
                                   precision=_PREC,
                                   preferred_element_type=jnp.float32)
        keep = 1.0 - jnp.sum(ot, axis=1, keepdims=True)
        if causal:
            rio2 = jax.lax.broadcasted_iota(jnp.int32, (CH, CH), 0)
            cio2 = jax.lax.broadcasted_iota(jnp.int32, (CH, CH), 1)
            tri = jnp.where(rio2 >= cio2, np.float32(1.0), np.float32(0.0))
            run = jnp.zeros((1, E), jnp.float32)
            for cix in range(L_K // CH):
                sl = slice(cix * CH, (cix + 1) * CH)
                cc = jax.lax.dot_general(tri, V[sl, :], (((1,), (0,)), ((), ())),
                                         precision=_PREC,
                                         preferred_element_type=jnp.float32)
                o_ref[0, sl, :] = (cc + run) * keep[sl, :] + scat[sl, :]
                run = run + cc[CH - 1:CH, :]
        else:
            vm = jnp.mean(V, axis=0, keepdims=True)
            o_ref[0] = jnp.broadcast_to(vm, (L_Q, E)) * keep + scat

    return pl.pallas_call(
        body,
        grid=(G,),
        in_specs=[pl.BlockSpec((1, 1, 128), lambda i: (i, 0, 0)),
                  pl.BlockSpec((1, L_Q, E), lambda i: (i, 0, 0)),
                  pl.BlockSpec((1, L_K, E), lambda i: (i, 0, 0)),
                  pl.BlockSpec((1, L_K, E), lambda i: (i, 0, 0))],
        out_specs=pl.BlockSpec((1, L_Q, E), lambda i: (i, 0, 0)),
        out_shape=jax.ShapeDtypeStruct((G, L_Q, E), jnp.float32),
    )(idx, Qh, Kh, Vh)


def _split_heads(x, B, L):
    return x.reshape(B, L, _H, _E).transpose(0, 2, 1, 3).reshape(B * _H, L, _E)


def _attention(pa, xq, xkv, fold, causal):
    B, L_Q, _ = xq.shape
    L_K = xkv.shape[1]
    q2 = xq.reshape(B * L_Q, _D)
    if xq is xkv:
        wqkv = jnp.concatenate([pa['qw'].T, pa['kw'].T, pa['vw'].T], axis=1)
        bqkv = jnp.concatenate([pa['qb'], pa['kb'], pa['vb']])
        qkv = _mm(q2, wqkv, bias=bqkv)
        Q, K, V = qkv[:, :_D], qkv[:, _D:2 * _D], qkv[:, 2 * _D:]
    else:
        kv2 = xkv.reshape(B * L_K, _D)
        Q = _mm(q2, pa['qw'].T, bias=pa['qb'])
        wkv = jnp.concatenate([pa['kw'].T, pa['vw'].T], axis=1)
        bkv = jnp.concatenate([pa['kb'], pa['vb']])
        kv = _mm(kv2, wkv, bias=bkv)
        K, V = kv[:, :_D], kv[:, _D:]
    Qh = _split_heads(Q, B, L_Q)
    Kh = _split_heads(K, B, L_K)
    Vh = _split_heads(V, B, L_K)
    C = jnp.asarray(_sample_counts(fold, L_Q, L_K))
    u = _u_of(L_Q)
    idx = _topk_queries(Qh, Kh, C, u)
    ctx = _attn_apply(idx, Qh, Kh, Vh, u, causal)
    ctx = ctx.reshape(B, _H, L_Q, _E).transpose(0, 2, 1, 3).reshape(B * L_Q, _D)
    return ctx


def _embed(x, x_mark, conv_w, temp_w):
    B, L, Cin = x.shape
    xcat = jnp.concatenate(
        [jnp.roll(x, 1, axis=1), x, jnp.roll(x, -1, axis=1), x_mark,
         jnp.zeros((B, L, 128 - 3 * Cin - _TF), jnp.float32)], axis=-1)
    W = jnp.concatenate(
        [conv_w[:, :, 0].T, conv_w[:, :, 1].T, conv_w[:, :, 2].T, temp_w.T,
         jnp.zeros((128 - 3 * Cin - _TF, _D), jnp.float32)], axis=0)
    pos = jnp.asarray(_pos_table(L))
    return _mm(xcat.reshape(B * L, 128), W, add=pos)


def _encoder_layer(p, x, fold):
    B, L, _ = x.shape
    x2 = x.reshape(B * L, _D)
    ctx = _attention(p['attn'], x, x, fold, causal=False)
    x1 = _mm(ctx, p['attn']['ow'].T, bias=p['attn']['ob'], res=x2,
             ln=(p['ln1g'], p['ln1b']))
    y = _mm(x1, p['c1w'].T, bias=p['c1b'], act="gelu")
    out = _mm(y, p['c2w'].T, bias=p['c2b'], res=x1, ln=(p['ln2g'], p['ln2b']))
    return out.reshape(B, L, _D)


def _distil(p, x):
    B, L, _ = x.shape
    xcat = jnp.concatenate(
        [jnp.roll(x, 1, axis=1), x, jnp.roll(x, -1, axis=1)], axis=-1)
    W = jnp.concatenate(
        [p['w'][:, :, 0].T, p['w'][:, :, 1].T, p['w'][:, :, 2].T], axis=0)
    y = _mm(xcat.reshape(B * L, 3 * _D), W, bias=p['b'],
            scale=1.0 / np.sqrt(1.0 + 1e-5), act="elu")
    y = y.reshape(B, L, _D)
    e = y[:, 0::2, :]
    o = y[:, 1::2, :]

    def body(e_ref, o_ref, out_ref):
        ev = e_ref[0]
        ov = o_ref[0]
        om1 = jnp.concatenate(
            [jnp.full((1, _D), -np.inf, jnp.float32), ov[:-1, :]], axis=0)
        out_ref[0] = jnp.maximum(jnp.maximum(ev, ov), om1)

    Lh = L // 2
    return pl.pallas_call(
        body,
        grid=(B,),
        in_specs=[pl.BlockSpec((1, Lh, _D), lambda i: (i, 0, 0)),
                  pl.BlockSpec((1, Lh, _D), lambda i: (i, 0, 0))],
        out_specs=pl.BlockSpec((1, Lh, _D), lambda i: (i, 0, 0)),
        out_shape=jax.ShapeDtypeStruct((B, Lh, _D), jnp.float32),
    )(e, o)


def _decoder_layer(p, x, cross, f1, f2):
    B, L, _ = x.shape
    Lc = cross.shape[1]
    x2 = x.reshape(B * L, _D)
    ctx = _attention(p['self'], x, x, f1, causal=True)
    x1 = _mm(ctx, p['self']['ow'].T, bias=p['self']['ob'], res=x2,
             ln=(p['ln1g'], p['ln1b']))
    ctx2 = _attention(p['cross'], x1.reshape(B, L, _D), cross, f2, causal=False)
    x2b = _mm(ctx2, p['cross']['ow'].T, bias=p['cross']['ob'], res=x1,
              ln=(p['ln2g'], p['ln2b']))
    y = _mm(x2b, p['c1w'].T, bias=p['c1b'], act="gelu")
    out = _mm(y, p['c2w'].T, bias=p['c2b'], res=x2b, ln=(p['ln3g'], p['ln3b']))
    return out.reshape(B, L, _D)


def kernel(x_enc, x_mark_enc, x_dec, x_mark_dec, params):
    p = params
    B, L_e, _ = x_enc.shape
    L_d = x_dec.shape[1]

    enc = _embed(x_enc, x_mark_enc, p['enc_conv_w'], p['enc_temp_w'])
    h = _encoder_layer(p['enc0'], enc.reshape(B, L_e, _D), 0)
    h = _distil(p['distil0'], h)
    Lh = h.shape[1]
    h = _encoder_layer(p['enc1'], h, 1)
    h2 = _layernorm(h.reshape(B * Lh, _D), p['enc_ng'], p['enc_nb'])
    h = h2.reshape(B, Lh, _D)

    dec = _embed(x_dec, x_mark_dec, p['dec_conv_w'], p['dec_temp_w'])
    d = _decoder_layer(p['dec0'], dec.reshape(B, L_d, _D), h, 2, 3)
    d2 = _layernorm(d.reshape(B * L_d, _D), p['dec_ng'], p['dec_nb'])

    d_last = d2.reshape(B, L_d, _D)[:, -_PRED_LEN:, :].reshape(B * _PRED_LEN, _D)
    Wp = jnp.concatenate(
        [p['proj_w'].T, jnp.zeros((_D, 128 - p['proj_w'].shape[0]), jnp.float32)],
        axis=1)
    bp = jnp.concatenate(
        [p['proj_b'], jnp.zeros((128 - p['proj_b'].shape[0],), jnp.float32)])
    out = _mm(d_last, Wp, bias=bp)
    return out[:, :p['proj_w'].shape[0]].reshape(B, _PRED_LEN, p['proj_w'].shape[0])


# speedup vs baseline: 1.0591x; 1.0591x over previous
import functools

import numpy as np
import jax
import jax.numpy as jnp
from jax.experimental import pallas as pl
from jax.experimental.pallas import tpu as pltpu

_D = 512
_H = 8
_E = 64
_FF = 2048
_TF = 4
_PRED_LEN = 1024
_PREC = jax.lax.Precision.DEFAULT


def _u_of(L):
    return min(int(5 * np.ceil(np.log(L))), L)


def _tf2x32(k0, k1, c0, c1):
    u32 = np.uint32
    rotations = [[13, 15, 26, 6], [17, 29, 16, 24]]
    ks = [u32(k0), u32(k1), u32(k0) ^ u32(k1) ^ u32(0x1BD11BDA)]
    x = [(c0 + ks[0]).astype(u32), (c1 + ks[1]).astype(u32)]
    for i in range(5):
        for r in rotations[i % 2]:
            x[0] = (x[0] + x[1]).astype(u32)
            x[1] = (((x[1] << u32(r)) | (x[1] >> u32(32 - r))).astype(u32)
                    ^ x[0])
        x[0] = (x[0] + ks[(i + 1) % 3]).astype(u32)
        x[1] = (x[1] + ks[(i + 2) % 3] + u32(i + 1)).astype(u32)
    return x[0], x[1]


def _np_randint(key, shape, maxval):
    o0, o1 = _tf2x32(key[0], key[1], np.zeros(2, np.uint32),
                     np.arange(2, dtype=np.uint32))
    k1 = np.array([o0[0], o1[0]], np.uint32)
    k2 = np.array([o0[1], o1[1]], np.uint32)

    def bits(k, n):
        b0, b1 = _tf2x32(k[0], k[1], np.zeros(n, np.uint32),
                         np.arange(n, dtype=np.uint32))
        return b0 ^ b1

    n = int(np.prod(shape))
    higher = bits(k1, n).reshape(shape)
    lower = bits(k2, n).reshape(shape)
    span = np.uint32(maxval)
    mult = ((np.uint32(65536) % span) * (np.uint32(65536) % span)) % span
    return (((higher % span) * mult + (lower % span)) % span).astype(np.int32)


@functools.lru_cache(maxsize=None)
def _sample_counts(fold, L_Q, L_K):
    key = _tf2x32(np.uint32(0), np.uint32(1234),
                  np.zeros(1, np.uint32), np.full(1, fold, np.uint32))
    key = np.array([key[0][0], key[1][0]], np.uint32)
    idx = _np_randint(key, (L_Q, _u_of(L_K)), L_K)
    C = np.zeros((L_Q, L_K), np.int8)
    np.add.at(C, (np.arange(L_Q)[:, None], idx), 1)
    return C


@functools.lru_cache(maxsize=None)
def _pos_table(L, d=_D):
    pos = np.arange(L, dtype=np.float32)[:, None]
    div = np.exp(np.arange(0, d, 2, dtype=np.float32) * (-np.log(10000.0) / d))
    pe = np.zeros((L, d), np.float32)
    pe[:, 0::2] = np.sin(pos * div)
    pe[:, 1::2] = np.cos(pos * div)
    return pe


def _mm(x, w, *, bias=None, add=None, add_period=None, res=None, ln=None,
        act=None, scale=1.0, blk=1024):
    M, K = x.shape
    N = w.shape[1]
    assert M % blk == 0, (M, blk)
    inputs = [x, w]
    specs = [pl.BlockSpec((blk, K), lambda i: (i, 0)),
             pl.BlockSpec((K, N), lambda i: (0, 0))]
    if bias is not None:
        inputs.append(bias.reshape(1, N))
        specs.append(pl.BlockSpec((1, N), lambda i: (0, 0)))
    if add is not None:
        nb = add.shape[0] // blk
        inputs.append(add)
        specs.append(pl.BlockSpec((blk, N), lambda i, _nb=nb: (i % _nb, 0)))
    if res is not None:
        inputs.append(res)
        specs.append(pl.BlockSpec((blk, N), lambda i: (i, 0)))
    if ln is not None:
        g, b = ln
        inputs += [g.reshape(1, N), b.reshape(1, N)]
        specs += [pl.BlockSpec((1, N), lambda i: (0, 0)),
                  pl.BlockSpec((1, N), lambda i: (0, 0))]

    def body(*refs):
        x_ref, w_ref, *rest = refs[:-1]
        o_ref = refs[-1]
        y = jax.lax.dot_general(x_ref[...], w_ref[...], (((1,), (0,)), ((), ())),
                                precision=_PREC,
                                preferred_element_type=jnp.float32)
        it = iter(rest)
        if bias is not None:
            y = y + next(it)[...]
        if add is not None:
            y = y + next(it)[...]
        if scale != 1.0:
            y = y * np.float32(scale)
        if act == "gelu":
            y = 0.5 * y * (1.0 + jax.lax.erf(y * np.float32(1.0 / np.sqrt(2.0))))
        elif act == "elu":
            y = jnp.where(y > 0, y, jnp.exp(jnp.minimum(y, 0.0)) - 1.0)
        if res is not None:
            y = y + next(it)[...]
        if ln is not None:
            g_ref = next(it)
            b_ref = next(it)
            m = jnp.mean(y, axis=-1, keepdims=True)
            d = y - m
            v = jnp.mean(d * d, axis=-1, keepdims=True)
            y = d * jax.lax.rsqrt(v + 1e-5) * g_ref[...] + b_ref[...]
        o_ref[...] = y

    return pl.pallas_call(
        body,
        grid=(M // blk,),
        in_specs=specs,
        out_specs=pl.BlockSpec((blk, N), lambda i: (i, 0)),
        out_shape=jax.ShapeDtypeStruct((M, N), jnp.float32),
    )(*inputs)


def _layernorm(x, g, b, blk=1024):
    M, N = x.shape

    def body(x_ref, g_ref, b_ref, o_ref):
        y = x_ref[...]
        m = jnp.mean(y, axis=-1, keepdims=True)
        d = y - m
        v = jnp.mean(d * d, axis=-1, keepdims=True)
        o_ref[...] = d * jax.lax.rsqrt(v + 1e-5) * g_ref[...] + b_ref[...]

    return pl.pallas_call(
        body,
        grid=(M // blk,),
        in_specs=[pl.BlockSpec((blk, N), lambda i: (i, 0)),
                  pl.BlockSpec((1, N), lambda i: (0, 0)),
                  pl.BlockSpec((1, N), lambda i: (0, 0))],
        out_specs=pl.BlockSpec((blk, N), lambda i: (i, 0)),
        out_shape=jax.ShapeDtypeStruct((M, N), jnp.float32),
    )(x, g.reshape(1, N), b.reshape(1, N))


def _topk_queries(Qh, Kh, C, u):
    G, L_Q, E = Qh.shape
    L_K = Kh.shape[1]
    QB = min(1024, L_Q)
    nj = L_Q // QB
    rows = QB // 128

    def body(q_ref, k_ref, c_ref, o_ref, m_scr):
        j = pl.program_id(1)
        qk = jax.lax.dot_general(q_ref[0], k_ref[0], (((1,), (1,)), ((), ())),
                                 precision=_PREC,
                                 preferred_element_type=jnp.float32)
        c = c_ref[pl.ds(j * QB, QB), :].astype(jnp.float32)
        mx = jnp.max(jnp.where(c > 0, qk, np.float32(-1e30)), axis=1)
        sm = jnp.sum(qk * c, axis=1) * np.float32(1.0 / L_K)
        m_scr[pl.ds(j * rows, rows), :] = (mx - sm).reshape(rows, 128)

        @pl.when(j == nj - 1)
        def _():
            Mv = m_scr[...]
            gio = (jax.lax.broadcasted_iota(jnp.int32, Mv.shape, 0) * 128
                   + jax.lax.broadcasted_iota(jnp.int32, Mv.shape, 1))
            lane = jax.lax.broadcasted_iota(jnp.int32, (1, 128), 1)
            out = jnp.full((1, 128), -1, jnp.int32)
            for t in range(u):
                mv = jnp.max(Mv)
                gi = jnp.min(jnp.where(Mv == mv, gio, jnp.int32(2**30)))
                out = jnp.where(lane == t, gi, out)
                Mv = jnp.where(gio == gi, np.float32(-np.inf), Mv)
            o_ref[0] = out

    idx = pl.pallas_call(
        body,
        grid=(G, nj),
        in_specs=[pl.BlockSpec((1, QB, E), lambda i, j: (i, j, 0)),
                  pl.BlockSpec((1, L_K, E), lambda i, j: (i, 0, 0)),
                  pl.BlockSpec((L_Q, L_K), lambda i, j: (0, 0))],
        out_specs=pl.BlockSpec((1, 1, 128), lambda i, j: (i, 0, 0)),
        out_shape=jax.ShapeDtypeStruct((G, 1, 128), jnp.int32),
        scratch_shapes=[pltpu.VMEM((L_Q // 128, 128), jnp.float32)],
    )(Qh, Kh, C)
    return idx


def _attn_apply(idx, Qh, Kh, Vh, u, causal):
    G, L_Q, E = Qh.shape
    L_K = Kh.shape[1]
    UP = 64
    CH = 512

    def body(idx_ref, q_ref, k_ref, v_ref, o_ref):
        V = v_ref[0]
        idp = idx_ref[0][:, :UP]
        rio = jax.lax.broadcasted_iota(jnp.int32, (L_Q, UP), 0)
        ot = jnp.where(rio == idp, np.float32(1.0), np.float32(0.0))
        qr = jax.lax.dot_general(ot, q_ref[0], (((0,), (0,)), ((), ())),
                                 precision=_PREC,
                                 preferred_element_type=jnp.float32)
        scores = jax.lax.dot_general(qr, k_ref[0], (((1,), (1,)), ((), ())),
                                     precision=_PREC,
                                     preferred_element_type=jnp.float32)
        scores = scores * np.float32(1.0 / np.sqrt(E))
        if causal:
            pcol = jnp.transpose(idp, (1, 0))
            kio = jax.lax.broadcasted_iota(jnp.int32, (UP, L_K), 1)
            scores = jnp.where(kio > pcol, np.float32(-1e30), scores)
        smax = jnp.max(scores, axis=-1, keepdims=True)
        ex = jnp.exp(scores - smax)
        attn = ex / jnp.sum(ex, axis=-1, keepdims=True)
        upd = jax.lax.dot_general(attn, V, (((1,), (0,)), ((), ())),
                                  precision=_PREC,
                                  preferred_element_type=jnp.float32)
        scat = jax.lax.dot_general(ot, upd, (((1,), (0,)), ((), ())),
                                   precision=_PREC,
                                   preferred_element_type=jnp.float32)
        keep = 1.0 - jnp.sum(ot, axis=1, keepdims=True)
        if causal:
            rio2 = jax.lax.broadcasted_iota(jnp.int32, (CH, CH), 0)
            cio2 = jax.lax.broadcasted_iota(jnp.int32, (CH, CH), 1)
            tri = jnp.where(rio2 >= cio2, np.float32(1.0), np.float32(0.0))
            run = jnp.zeros((1, E), jnp.float32)
            for cix in range(L_K // CH):
                sl = slice(cix * CH, (cix + 1) * CH)
                cc = jax.lax.dot_general(tri, V[sl, :], (((1,), (0,)), ((), ())),
                                         precision=_PREC,
                                         preferred_element_type=jnp.float32)
                o_ref[0, sl, :] = (cc + run) * keep[sl, :] + scat[sl, :]
                run = run + cc[CH - 1:CH, :]
        else:
            vm = jnp.mean(V, axis=0, keepdims=True)
            o_ref[0] = jnp.broadcast_to(vm, (L_Q, E)) * keep + scat

    return pl.pallas_call(
        body,
        grid=(G,),
        in_specs=[pl.BlockSpec((1, 1, 128), lambda i: (i, 0, 0)),
                  pl.BlockSpec((1, L_Q, E), lambda i: (i, 0, 0)),
                  pl.BlockSpec((1, L_K, E), lambda i: (i, 0, 0)),
                  pl.BlockSpec((1, L_K, E), lambda i: (i, 0, 0))],
        out_specs=pl.BlockSpec((1, L_Q, E), lambda i: (i, 0, 0)),
        out_shape=jax.ShapeDtypeStruct((G, L_Q, E), jnp.float32),
    )(idx, Qh, Kh, Vh)


def _split_heads(x, B, L):
    return x.reshape(B, L, _H, _E).transpose(0, 2, 1, 3).reshape(B * _H, L, _E)


def _attention(pa, xq, xkv, fold, causal):
    B, L_Q, _ = xq.shape
    L_K = xkv.shape[1]
    q2 = xq.reshape(B * L_Q, _D)
    if xq is xkv:
        wqkv = jnp.concatenate([pa['qw'].T, pa['kw'].T, pa['vw'].T], axis=1)
        bqkv = jnp.concatenate([pa['qb'], pa['kb'], pa['vb']])
        qkv = _mm(q2, wqkv, bias=bqkv)
        Q, K, V = qkv[:, :_D], qkv[:, _D:2 * _D], qkv[:, 2 * _D:]
    else:
        kv2 = xkv.reshape(B * L_K, _D)
        Q = _mm(q2, pa['qw'].T, bias=pa['qb'])
        wkv = jnp.concatenate([pa['kw'].T, pa['vw'].T], axis=1)
        bkv = jnp.concatenate([pa['kb'], pa['vb']])
        kv = _mm(kv2, wkv, bias=bkv)
        K, V = kv[:, :_D], kv[:, _D:]
    Qh = _split_heads(Q, B, L_Q)
    Kh = _split_heads(K, B, L_K)
    Vh = _split_heads(V, B, L_K)
    C = jnp.asarray(_sample_counts(fold, L_Q, L_K))
    u = _u_of(L_Q)
    idx = _topk_queries(Qh, Kh, C, u)
    ctx = _attn_apply(idx, Qh, Kh, Vh, u, causal)
    ctx = ctx.reshape(B, _H, L_Q, _E).transpose(0, 2, 1, 3).reshape(B * L_Q, _D)
    return ctx


def _embed(x, x_mark, conv_w, temp_w):
    B, L, Cin = x.shape
    xcat = jnp.concatenate(
        [jnp.roll(x, 1, axis=1), x, jnp.roll(x, -1, axis=1), x_mark,
         jnp.zeros((B, L, 128 - 3 * Cin - _TF), jnp.float32)], axis=-1)
    W = jnp.concatenate(
        [conv_w[:, :, 0].T, conv_w[:, :, 1].T, conv_w[:, :, 2].T, temp_w.T,
         jnp.zeros((128 - 3 * Cin - _TF, _D), jnp.float32)], axis=0)
    pos = jnp.asarray(_pos_table(L))
    return _mm(xcat.reshape(B * L, 128), W, add=pos)


def _encoder_layer(p, x, fold):
    B, L, _ = x.shape
    x2 = x.reshape(B * L, _D)
    ctx = _attention(p['attn'], x, x, fold, causal=False)
    x1 = _mm(ctx, p['attn']['ow'].T, bias=p['attn']['ob'], res=x2,
             ln=(p['ln1g'], p['ln1b']))
    y = _mm(x1, p['c1w'].T, bias=p['c1b'], act="gelu")
    out = _mm(y, p['c2w'].T, bias=p['c2b'], res=x1, ln=(p['ln2g'], p['ln2b']))
    return out.reshape(B, L, _D)


def _distil(p, x):
    B, L, _ = x.shape
    xcat = jnp.concatenate(
        [jnp.roll(x, 1, axis=1), x, jnp.roll(x, -1, axis=1)], axis=-1)
    W = jnp.concatenate(
        [p['w'][:, :, 0].T, p['w'][:, :, 1].T, p['w'][:, :, 2].T], axis=0)
    y = _mm(xcat.reshape(B * L, 3 * _D), W, bias=p['b'],
            scale=1.0 / np.sqrt(1.0 + 1e-5), act="elu")
    y = y.reshape(B, L, _D)
    e = y[:, 0::2, :]
    o = y[:, 1::2, :]

    def body(e_ref, o_ref, out_ref):
        ev = e_ref[0]
        ov = o_ref[0]
        om1 = jnp.concatenate(
            [jnp.full((1, _D), -np.inf, jnp.float32), ov[:-1, :]], axis=0)
        out_ref[0] = jnp.maximum(jnp.maximum(ev, ov), om1)

    Lh = L // 2
    return pl.pallas_call(
        body,
        grid=(B,),
        in_specs=[pl.BlockSpec((1, Lh, _D), lambda i: (i, 0, 0)),
                  pl.BlockSpec((1, Lh, _D), lambda i: (i, 0, 0))],
        out_specs=pl.BlockSpec((1, Lh, _D), lambda i: (i, 0, 0)),
        out_shape=jax.ShapeDtypeStruct((B, Lh, _D), jnp.float32),
    )(e, o)


def _decoder_layer(p, x, cross, f1, f2):
    B, L, _ = x.shape
    Lc = cross.shape[1]
    x2 = x.reshape(B * L, _D)
    ctx = _attention(p['self'], x, x, f1, causal=True)
    x1 = _mm(ctx, p['self']['ow'].T, bias=p['self']['ob'], res=x2,
             ln=(p['ln1g'], p['ln1b']))
    ctx2 = _attention(p['cross'], x1.reshape(B, L, _D), cross, f2, causal=False)
    x2b = _mm(ctx2, p['cross']['ow'].T, bias=p['cross']['ob'], res=x1,
              ln=(p['ln2g'], p['ln2b']))
    y = _mm(x2b, p['c1w'].T, bias=p['c1b'], act="gelu")
    out = _mm(y, p['c2w'].T, bias=p['c2b'], res=x2b, ln=(p['ln3g'], p['ln3b']))
    return out.reshape(B, L, _D)


def kernel(x_enc, x_mark_enc, x_dec, x_mark_dec, params):
    p = params
    B, L_e, _ = x_enc.shape
    L_d = x_dec.shape[1]

    enc = _embed(x_enc, x_mark_enc, p['enc_conv_w'], p['enc_temp_w'])
    h = _encoder_layer(p['enc0'], enc.reshape(B, L_e, _D), 0)
    h = _distil(p['distil0'], h)
    Lh = h.shape[1]
    h = _encoder_layer(p['enc1'], h, 1)
    h2 = _layernorm(h.reshape(B * Lh, _D), p['enc_ng'], p['enc_nb'])
    h = h2.reshape(B, Lh, _D)

    dec = _embed(x_dec, x_mark_dec, p['dec_conv_w'], p['dec_temp_w'])
    d = _decoder_layer(p['dec0'], dec.reshape(B, L_d, _D), h, 2, 3)
    d2 = _layernorm(d.reshape(B * L_d, _D), p['dec_ng'], p['dec_nb'])

    d_last = d2.reshape(B, L_d, _D)[:, -_PRED_LEN:, :].reshape(B * _PRED_LEN, _D)
    Wp = jnp.concatenate(
        [p['proj_w'].T, jnp.zeros((_D, 128 - p['proj_w'].shape[0]), jnp.float32)],
        axis=1)
    bp = jnp.concatenate(
        [p['proj_b'], jnp.zeros((128 - p['proj_b'].shape[0],), jnp.float32)])
    out = _mm(d_last, Wp, bias=bp)
    return out[:, :p['proj_w'].shape[0]].reshape(B, _PRED_LEN, p['proj_w'].shape[0])


# speedup vs baseline: 1.7963x; 1.6960x over previous
import functools

import numpy as np
import jax
import jax.numpy as jnp
from jax.experimental import pallas as pl
from jax.experimental.pallas import tpu as pltpu

_D = 512
_H = 8
_E = 64
_FF = 2048
_TF = 4
_PRED_LEN = 1024
_PREC = jax.lax.Precision.DEFAULT


def _u_of(L):
    return min(int(5 * np.ceil(np.log(L))), L)


def _tf2x32(k0, k1, c0, c1):
    u32 = np.uint32
    rotations = [[13, 15, 26, 6], [17, 29, 16, 24]]
    ks = [u32(k0), u32(k1), u32(k0) ^ u32(k1) ^ u32(0x1BD11BDA)]
    x = [(c0 + ks[0]).astype(u32), (c1 + ks[1]).astype(u32)]
    for i in range(5):
        for r in rotations[i % 2]:
            x[0] = (x[0] + x[1]).astype(u32)
            x[1] = (((x[1] << u32(r)) | (x[1] >> u32(32 - r))).astype(u32)
                    ^ x[0])
        x[0] = (x[0] + ks[(i + 1) % 3]).astype(u32)
        x[1] = (x[1] + ks[(i + 2) % 3] + u32(i + 1)).astype(u32)
    return x[0], x[1]


def _np_randint(key, shape, maxval):
    o0, o1 = _tf2x32(key[0], key[1], np.zeros(2, np.uint32),
                     np.arange(2, dtype=np.uint32))
    k1 = np.array([o0[0], o1[0]], np.uint32)
    k2 = np.array([o0[1], o1[1]], np.uint32)

    def bits(k, n):
        b0, b1 = _tf2x32(k[0], k[1], np.zeros(n, np.uint32),
                         np.arange(n, dtype=np.uint32))
        return b0 ^ b1

    n = int(np.prod(shape))
    higher = bits(k1, n).reshape(shape)
    lower = bits(k2, n).reshape(shape)
    span = np.uint32(maxval)
    mult = ((np.uint32(65536) % span) * (np.uint32(65536) % span)) % span
    return (((higher % span) * mult + (lower % span)) % span).astype(np.int32)


@functools.lru_cache(maxsize=None)
def _sample_counts(fold, L_Q, L_K):
    key = _tf2x32(np.uint32(0), np.uint32(1234),
                  np.zeros(1, np.uint32), np.full(1, fold, np.uint32))
    key = np.array([key[0][0], key[1][0]], np.uint32)
    idx = _np_randint(key, (L_Q, _u_of(L_K)), L_K)
    C = np.zeros((L_Q, L_K), np.int8)
    np.add.at(C, (np.arange(L_Q)[:, None], idx), 1)
    return C


@functools.lru_cache(maxsize=None)
def _pos_table(L, d=_D):
    pos = np.arange(L, dtype=np.float32)[:, None]
    div = np.exp(np.arange(0, d, 2, dtype=np.float32) * (-np.log(10000.0) / d))
    pe = np.zeros((L, d), np.float32)
    pe[:, 0::2] = np.sin(pos * div)
    pe[:, 1::2] = np.cos(pos * div)
    return pe


def _mm(x, w, *, bias=None, add=None, add_period=None, res=None, ln=None,
        act=None, scale=1.0, blk=1024):
    M, K = x.shape
    N = w.shape[1]
    assert M % blk == 0, (M, blk)
    inputs = [x, w]
    specs = [pl.BlockSpec((blk, K), lambda i: (i, 0)),
             pl.BlockSpec((K, N), lambda i: (0, 0))]
    if bias is not None:
        inputs.append(bias.reshape(1, N))
        specs.append(pl.BlockSpec((1, N), lambda i: (0, 0)))
    if add is not None:
        nb = add.shape[0] // blk
        inputs.append(add)
        specs.append(pl.BlockSpec((blk, N), lambda i, _nb=nb: (i % _nb, 0)))
    if res is not None:
        inputs.append(res)
        specs.append(pl.BlockSpec((blk, N), lambda i: (i, 0)))
    if ln is not None:
        g, b = ln
        inputs += [g.reshape(1, N), b.reshape(1, N)]
        specs += [pl.BlockSpec((1, N), lambda i: (0, 0)),
                  pl.BlockSpec((1, N), lambda i: (0, 0))]

    def body(*refs):
        x_ref, w_ref, *rest = refs[:-1]
        o_ref = refs[-1]
        y = jax.lax.dot_general(x_ref[...], w_ref[...], (((1,), (0,)), ((), ())),
                                precision=_PREC,
                                preferred_element_type=jnp.float32)
        it = iter(rest)
        if bias is not None:
            y = y + next(it)[...]
        if add is not None:
            y = y + next(it)[...]
        if scale != 1.0:
            y = y * np.float32(scale)
        if act == "gelu":
            y = 0.5 * y * (1.0 + jax.lax.erf(y * np.float32(1.0 / np.sqrt(2.0))))
        elif act == "elu":
            y = jnp.where(y > 0, y, jnp.exp(jnp.minimum(y, 0.0)) - 1.0)
        if res is not None:
            y = y + next(it)[...]
        if ln is not None:
            g_ref = next(it)
            b_ref = next(it)
            m = jnp.mean(y, axis=-1, keepdims=True)
            d = y - m
            v = jnp.mean(d * d, axis=-1, keepdims=True)
            y = d * jax.lax.rsqrt(v + 1e-5) * g_ref[...] + b_ref[...]
        o_ref[...] = y

    return pl.pallas_call(
        body,
        grid=(M // blk,),
        in_specs=specs,
        out_specs=pl.BlockSpec((blk, N), lambda i: (i, 0)),
        out_shape=jax.ShapeDtypeStruct((M, N), jnp.float32),
    )(*inputs)


def _layernorm(x, g, b, blk=1024):
    M, N = x.shape

    def body(x_ref, g_ref, b_ref, o_ref):
        y = x_ref[...]
        m = jnp.mean(y, axis=-1, keepdims=True)
        d = y - m
        v = jnp.mean(d * d, axis=-1, keepdims=True)
        o_ref[...] = d * jax.lax.rsqrt(v + 1e-5) * g_ref[...] + b_ref[...]

    return pl.pallas_call(
        body,
        grid=(M // blk,),
        in_specs=[pl.BlockSpec((blk, N), lambda i: (i, 0)),
                  pl.BlockSpec((1, N), lambda i: (0, 0)),
                  pl.BlockSpec((1, N), lambda i: (0, 0))],
        out_specs=pl.BlockSpec((blk, N), lambda i: (i, 0)),
        out_shape=jax.ShapeDtypeStruct((M, N), jnp.float32),
    )(x, g.reshape(1, N), b.reshape(1, N))


def _topk_queries(Qh, Kh, C, u):
    G, L_Q, E = Qh.shape
    L_K = Kh.shape[1]
    QB = min(1024, L_Q)
    nj = L_Q // QB
    rows = QB // 128

    def mbody(q_ref, k_ref, c_ref, o_ref):
        j = pl.program_id(1)
        qk = jax.lax.dot_general(q_ref[0], k_ref[0], (((1,), (1,)), ((), ())),
                                 precision=_PREC,
                                 preferred_element_type=jnp.float32)
        c = c_ref[pl.ds(j * QB, QB), :].astype(jnp.float32)
        mx = jnp.max(jnp.where(c > 0, qk, np.float32(-1e30)), axis=1)
        sm = jnp.sum(qk * c, axis=1) * np.float32(1.0 / L_K)
        o_ref[0] = (mx - sm).reshape(rows, 128)

    M = pl.pallas_call(
        mbody,
        grid=(G, nj),
        in_specs=[pl.BlockSpec((1, QB, E), lambda i, j: (i, j, 0)),
                  pl.BlockSpec((1, L_K, E), lambda i, j: (i, 0, 0)),
                  pl.BlockSpec((L_Q, L_K), lambda i, j: (0, 0))],
        out_specs=pl.BlockSpec((1, rows, 128), lambda i, j: (i, j, 0)),
        out_shape=jax.ShapeDtypeStruct((G, L_Q // 128, 128), jnp.float32),
    )(Qh, Kh, C)

    def kbody(m_ref, o_ref):
        Mv = m_ref[...]
        gio = (jax.lax.broadcasted_iota(jnp.int32, Mv.shape, 1) * 128
               + jax.lax.broadcasted_iota(jnp.int32, Mv.shape, 2))
        lane = jax.lax.broadcasted_iota(jnp.int32, (G, 128), 1)
        out = jnp.full((G, 128), -1, jnp.int32)
        for t in range(u):
            mv = jnp.max(Mv, axis=(1, 2), keepdims=True)
            gi = jnp.min(jnp.where(Mv == mv, gio, jnp.int32(2**30)),
                         axis=(1, 2), keepdims=True)
            out = jnp.where(lane == t, gi[:, :, 0], out)
            Mv = jnp.where(gio == gi, np.float32(-np.inf), Mv)
        o_ref[...] = out.reshape(G, 1, 128)

    return pl.pallas_call(
        kbody,
        out_shape=jax.ShapeDtypeStruct((G, 1, 128), jnp.int32),
    )(M)


def _attn_apply(idx, Qh, Kh, Vh, u, causal):
    G, L_Q, E = Qh.shape
    L_K = Kh.shape[1]
    UP = 64
    CH = 512

    def body(idx_ref, q_ref, k_ref, v_ref, o_ref):
        V = v_ref[0]
        idp = idx_ref[0][:, :UP]
        rio = jax.lax.broadcasted_iota(jnp.int32, (L_Q, UP), 0)
        ot = jnp.where(rio == idp, np.float32(1.0), np.float32(0.0))
        qr = jax.lax.dot_general(ot, q_ref[0], (((0,), (0,)), ((), ())),
                                 precision=_PREC,
                                 preferred_element_type=jnp.float32)
        scores = jax.lax.dot_general(qr, k_ref[0], (((1,), (1,)), ((), ())),
                                     precision=_PREC,
                                     preferred_element_type=jnp.float32)
        scores = scores * np.float32(1.0 / np.sqrt(E))
        if causal:
            pcol = jnp.transpose(idp, (1, 0))
            kio = jax.lax.broadcasted_iota(jnp.int32, (UP, L_K), 1)
            scores = jnp.where(kio > pcol, np.float32(-1e30), scores)
        smax = jnp.max(scores, axis=-1, keepdims=True)
        ex = jnp.exp(scores - smax)
        attn = ex / jnp.sum(ex, axis=-1, keepdims=True)
        upd = jax.lax.dot_general(attn, V, (((1,), (0,)), ((), ())),
                                  precision=_PREC,
                                  preferred_element_type=jnp.float32)
        scat = jax.lax.dot_general(ot, upd, (((1,), (0,)), ((), ())),
                                   precision=_PREC,
                                   preferred_element_type=jnp.float32)
        keep = 1.0 - jnp.sum(ot, axis=1, keepdims=True)
        if causal:
            rio2 = jax.lax.broadcasted_iota(jnp.int32, (CH, CH), 0)
            cio2 = jax.lax.broadcasted_iota(jnp.int32, (CH, CH), 1)
            tri = jnp.where(rio2 >= cio2, np.float32(1.0), np.float32(0.0))
            run = jnp.zeros((1, E), jnp.float32)
            for cix in range(L_K // CH):
                sl = slice(cix * CH, (cix + 1) * CH)
                cc = jax.lax.dot_general(tri, V[sl, :], (((1,), (0,)), ((), ())),
                                         precision=_PREC,
                                         preferred_element_type=jnp.float32)
                o_ref[0, sl, :] = (cc + run) * keep[sl, :] + scat[sl, :]
                run = run + cc[CH - 1:CH, :]
        else:
            vm = jnp.mean(V, axis=0, keepdims=True)
            o_ref[0] = jnp.broadcast_to(vm, (L_Q, E)) * keep + scat

    return pl.pallas_call(
        body,
        grid=(G,),
        in_specs=[pl.BlockSpec((1, 1, 128), lambda i: (i, 0, 0)),
                  pl.BlockSpec((1, L_Q, E), lambda i: (i, 0, 0)),
                  pl.BlockSpec((1, L_K, E), lambda i: (i, 0, 0)),
                  pl.BlockSpec((1, L_K, E), lambda i: (i, 0, 0))],
        out_specs=pl.BlockSpec((1, L_Q, E), lambda i: (i, 0, 0)),
        out_shape=jax.ShapeDtypeStruct((G, L_Q, E), jnp.float32),
    )(idx, Qh, Kh, Vh)


def _split_heads(x, B, L):
    return x.reshape(B, L, _H, _E).transpose(0, 2, 1, 3).reshape(B * _H, L, _E)


def _attention(pa, xq, xkv, fold, causal):
    B, L_Q, _ = xq.shape
    L_K = xkv.shape[1]
    q2 = xq.reshape(B * L_Q, _D)
    if xq is xkv:
        wqkv = jnp.concatenate([pa['qw'].T, pa['kw'].T, pa['vw'].T], axis=1)
        bqkv = jnp.concatenate([pa['qb'], pa['kb'], pa['vb']])
        qkv = _mm(q2, wqkv, bias=bqkv)
        Q, K, V = qkv[:, :_D], qkv[:, _D:2 * _D], qkv[:, 2 * _D:]
    else:
        kv2 = xkv.reshape(B * L_K, _D)
        Q = _mm(q2, pa['qw'].T, bias=pa['qb'])
        wkv = jnp.concatenate([pa['kw'].T, pa['vw'].T], axis=1)
        bkv = jnp.concatenate([pa['kb'], pa['vb']])
        kv = _mm(kv2, wkv, bias=bkv)
        K, V = kv[:, :_D], kv[:, _D:]
    Qh = _split_heads(Q, B, L_Q)
    Kh = _split_heads(K, B, L_K)
    Vh = _split_heads(V, B, L_K)
    C = jnp.asarray(_sample_counts(fold, L_Q, L_K))
    u = _u_of(L_Q)
    idx = _topk_queries(Qh, Kh, C, u)
    ctx = _attn_apply(idx, Qh, Kh, Vh, u, causal)
    ctx = ctx.reshape(B, _H, L_Q, _E).transpose(0, 2, 1, 3).reshape(B * L_Q, _D)
    return ctx


def _embed(x, x_mark, conv_w, temp_w):
    B, L, Cin = x.shape
    xcat = jnp.concatenate(
        [jnp.roll(x, 1, axis=1), x, jnp.roll(x, -1, axis=1), x_mark,
         jnp.zeros((B, L, 128 - 3 * Cin - _TF), jnp.float32)], axis=-1)
    W = jnp.concatenate(
        [conv_w[:, :, 0].T, conv_w[:, :, 1].T, conv_w[:, :, 2].T, temp_w.T,
         jnp.zeros((128 - 3 * Cin - _TF, _D), jnp.float32)], axis=0)
    pos = jnp.asarray(_pos_table(L))
    return _mm(xcat.reshape(B * L, 128), W, add=pos)


def _encoder_layer(p, x, fold):
    B, L, _ = x.shape
    x2 = x.reshape(B * L, _D)
    ctx = _attention(p['attn'], x, x, fold, causal=False)
    x1 = _mm(ctx, p['attn']['ow'].T, bias=p['attn']['ob'], res=x2,
             ln=(p['ln1g'], p['ln1b']))
    y = _mm(x1, p['c1w'].T, bias=p['c1b'], act="gelu")
    out = _mm(y, p['c2w'].T, bias=p['c2b'], res=x1, ln=(p['ln2g'], p['ln2b']))
    return out.reshape(B, L, _D)


def _distil(p, x):
    B, L, _ = x.shape
    xcat = jnp.concatenate(
        [jnp.roll(x, 1, axis=1), x, jnp.roll(x, -1, axis=1)], axis=-1)
    W = jnp.concatenate(
        [p['w'][:, :, 0].T, p['w'][:, :, 1].T, p['w'][:, :, 2].T], axis=0)
    y = _mm(xcat.reshape(B * L, 3 * _D), W, bias=p['b'],
            scale=1.0 / np.sqrt(1.0 + 1e-5), act="elu")
    y = y.reshape(B, L, _D)
    e = y[:, 0::2, :]
    o = y[:, 1::2, :]

    def body(e_ref, o_ref, out_ref):
        ev = e_ref[0]
        ov = o_ref[0]
        om1 = jnp.concatenate(
            [jnp.full((1, _D), -np.inf, jnp.float32), ov[:-1, :]], axis=0)
        out_ref[0] = jnp.maximum(jnp.maximum(ev, ov), om1)

    Lh = L // 2
    return pl.pallas_call(
        body,
        grid=(B,),
        in_specs=[pl.BlockSpec((1, Lh, _D), lambda i: (i, 0, 0)),
                  pl.BlockSpec((1, Lh, _D), lambda i: (i, 0, 0))],
        out_specs=pl.BlockSpec((1, Lh, _D), lambda i: (i, 0, 0)),
        out_shape=jax.ShapeDtypeStruct((B, Lh, _D), jnp.float32),
    )(e, o)


def _decoder_layer(p, x, cross, f1, f2):
    B, L, _ = x.shape
    Lc = cross.shape[1]
    x2 = x.reshape(B * L, _D)
    ctx = _attention(p['self'], x, x, f1, causal=True)
    x1 = _mm(ctx, p['self']['ow'].T, bias=p['self']['ob'], res=x2,
             ln=(p['ln1g'], p['ln1b']))
    ctx2 = _attention(p['cross'], x1.reshape(B, L, _D), cross, f2, causal=False)
    x2b = _mm(ctx2, p['cross']['ow'].T, bias=p['cross']['ob'], res=x1,
              ln=(p['ln2g'], p['ln2b']))
    y = _mm(x2b, p['c1w'].T, bias=p['c1b'], act="gelu")
    out = _mm(y, p['c2w'].T, bias=p['c2b'], res=x2b, ln=(p['ln3g'], p['ln3b']))
    return out.reshape(B, L, _D)


def kernel(x_enc, x_mark_enc, x_dec, x_mark_dec, params):
    p = params
    B, L_e, _ = x_enc.shape
    L_d = x_dec.shape[1]

    enc = _embed(x_enc, x_mark_enc, p['enc_conv_w'], p['enc_temp_w'])
    h = _encoder_layer(p['enc0'], enc.reshape(B, L_e, _D), 0)
    h = _distil(p['distil0'], h)
    Lh = h.shape[1]
    h = _encoder_layer(p['enc1'], h, 1)
    h2 = _layernorm(h.reshape(B * Lh, _D), p['enc_ng'], p['enc_nb'])
    h = h2.reshape(B, Lh, _D)

    dec = _embed(x_dec, x_mark_dec, p['dec_conv_w'], p['dec_temp_w'])
    d = _decoder_layer(p['dec0'], dec.reshape(B, L_d, _D), h, 2, 3)
    d2 = _layernorm(d.reshape(B * L_d, _D), p['dec_ng'], p['dec_nb'])

    d_last = d2.reshape(B, L_d, _D)[:, -_PRED_LEN:, :].reshape(B * _PRED_LEN, _D)
    Wp = jnp.concatenate(
        [p['proj_w'].T, jnp.zeros((_D, 128 - p['proj_w'].shape[0]), jnp.float32)],
        axis=1)
    bp = jnp.concatenate(
        [p['proj_b'], jnp.zeros((128 - p['proj_b'].shape[0],), jnp.float32)])
    out = _mm(d_last, Wp, bias=bp)
    return out[:, :p['proj_w'].shape[0]].reshape(B, _PRED_LEN, p['proj_w'].shape[0])


# speedup vs baseline: 2.4768x; 1.3788x over previous
import functools

import numpy as np
import jax
import jax.numpy as jnp
from jax.experimental import pallas as pl
from jax.experimental.pallas import tpu as pltpu

_D = 512
_H = 8
_E = 64
_FF = 2048
_TF = 4
_PRED_LEN = 1024
_PREC = jax.lax.Precision.DEFAULT


def _u_of(L):
    return min(int(5 * np.ceil(np.log(L))), L)


def _tf2x32(k0, k1, c0, c1):
    u32 = np.uint32
    rotations = [[13, 15, 26, 6], [17, 29, 16, 24]]
    ks = [u32(k0), u32(k1), u32(k0) ^ u32(k1) ^ u32(0x1BD11BDA)]
    x = [(c0 + ks[0]).astype(u32), (c1 + ks[1]).astype(u32)]
    for i in range(5):
        for r in rotations[i % 2]:
            x[0] = (x[0] + x[1]).astype(u32)
            x[1] = (((x[1] << u32(r)) | (x[1] >> u32(32 - r))).astype(u32)
                    ^ x[0])
        x[0] = (x[0] + ks[(i + 1) % 3]).astype(u32)
        x[1] = (x[1] + ks[(i + 2) % 3] + u32(i + 1)).astype(u32)
    return x[0], x[1]


def _np_randint(key, shape, maxval):
    o0, o1 = _tf2x32(key[0], key[1], np.zeros(2, np.uint32),
                     np.arange(2, dtype=np.uint32))
    k1 = np.array([o0[0], o1[0]], np.uint32)
    k2 = np.array([o0[1], o1[1]], np.uint32)

    def bits(k, n):
        b0, b1 = _tf2x32(k[0], k[1], np.zeros(n, np.uint32),
                         np.arange(n, dtype=np.uint32))
        return b0 ^ b1

    n = int(np.prod(shape))
    higher = bits(k1, n).reshape(shape)
    lower = bits(k2, n).reshape(shape)
    span = np.uint32(maxval)
    mult = ((np.uint32(65536) % span) * (np.uint32(65536) % span)) % span
    return (((higher % span) * mult + (lower % span)) % span).astype(np.int32)


@functools.lru_cache(maxsize=None)
def _sample_counts(fold, L_Q, L_K):
    key = _tf2x32(np.uint32(0), np.uint32(1234),
                  np.zeros(1, np.uint32), np.full(1, fold, np.uint32))
    key = np.array([key[0][0], key[1][0]], np.uint32)
    idx = _np_randint(key, (L_Q, _u_of(L_K)), L_K)
    C = np.zeros((L_Q, L_K), np.int8)
    np.add.at(C, (np.arange(L_Q)[:, None], idx), 1)
    return C


@functools.lru_cache(maxsize=None)
def _pos_table(L, d=_D):
    pos = np.arange(L, dtype=np.float32)[:, None]
    div = np.exp(np.arange(0, d, 2, dtype=np.float32) * (-np.log(10000.0) / d))
    pe = np.zeros((L, d), np.float32)
    pe[:, 0::2] = np.sin(pos * div)
    pe[:, 1::2] = np.cos(pos * div)
    return pe


def _mm(x, w, *, bias=None, add=None, add_period=None, res=None, ln=None,
        act=None, scale=1.0, blk=1024):
    M, K = x.shape
    N = w.shape[1]
    assert M % blk == 0, (M, blk)
    inputs = [x, w]
    specs = [pl.BlockSpec((blk, K), lambda i: (i, 0)),
             pl.BlockSpec((K, N), lambda i: (0, 0))]
    if bias is not None:
        inputs.append(bias.reshape(1, N))
        specs.append(pl.BlockSpec((1, N), lambda i: (0, 0)))
    if add is not None:
        nb = add.shape[0] // blk
        inputs.append(add)
        specs.append(pl.BlockSpec((blk, N), lambda i, _nb=nb: (i % _nb, 0)))
    if res is not None:
        inputs.append(res)
        specs.append(pl.BlockSpec((blk, N), lambda i: (i, 0)))
    if ln is not None:
        g, b = ln
        inputs += [g.reshape(1, N), b.reshape(1, N)]
        specs += [pl.BlockSpec((1, N), lambda i: (0, 0)),
                  pl.BlockSpec((1, N), lambda i: (0, 0))]

    def body(*refs):
        x_ref, w_ref, *rest = refs[:-1]
        o_ref = refs[-1]
        y = jax.lax.dot_general(x_ref[...], w_ref[...], (((1,), (0,)), ((), ())),
                                precision=_PREC,
                                preferred_element_type=jnp.float32)
        it = iter(rest)
        if bias is not None:
            y = y + next(it)[...]
        if add is not None:
            y = y + next(it)[...]
        if scale != 1.0:
            y = y * np.float32(scale)
        if act == "gelu":
            y = 0.5 * y * (1.0 + jax.lax.erf(y * np.float32(1.0 / np.sqrt(2.0))))
        elif act == "elu":
            y = jnp.where(y > 0, y, jnp.exp(jnp.minimum(y, 0.0)) - 1.0)
        if res is not None:
            y = y + next(it)[...]
        if ln is not None:
            g_ref = next(it)
            b_ref = next(it)
            m = jnp.mean(y, axis=-1, keepdims=True)
            d = y - m
            v = jnp.mean(d * d, axis=-1, keepdims=True)
            y = d * jax.lax.rsqrt(v + 1e-5) * g_ref[...] + b_ref[...]
        o_ref[...] = y

    return pl.pallas_call(
        body,
        grid=(M // blk,),
        in_specs=specs,
        out_specs=pl.BlockSpec((blk, N), lambda i: (i, 0)),
        out_shape=jax.ShapeDtypeStruct((M, N), jnp.float32),
    )(*inputs)


def _layernorm(x, g, b, blk=1024):
    M, N = x.shape

    def body(x_ref, g_ref, b_ref, o_ref):
        y = x_ref[...]
        m = jnp.mean(y, axis=-1, keepdims=True)
        d = y - m
        v = jnp.mean(d * d, axis=-1, keepdims=True)
        o_ref[...] = d * jax.lax.rsqrt(v + 1e-5) * g_ref[...] + b_ref[...]

    return pl.pallas_call(
        body,
        grid=(M // blk,),
        in_specs=[pl.BlockSpec((blk, N), lambda i: (i, 0)),
                  pl.BlockSpec((1, N), lambda i: (0, 0)),
                  pl.BlockSpec((1, N), lambda i: (0, 0))],
        out_specs=pl.BlockSpec((blk, N), lambda i: (i, 0)),
        out_shape=jax.ShapeDtypeStruct((M, N), jnp.float32),
    )(x, g.reshape(1, N), b.reshape(1, N))


def _topk_queries(Qh, Kh, C, u):
    G, L_Q, E = Qh.shape
    L_K = Kh.shape[1]
    QB = min(1024, L_Q)
    nj = L_Q // QB
    rows = QB // 128

    def mbody(q_ref, k_ref, c_ref, o_ref):
        j = pl.program_id(1)
        qk = jax.lax.dot_general(q_ref[0], k_ref[0], (((1,), (1,)), ((), ())),
                                 precision=_PREC,
                                 preferred_element_type=jnp.float32)
        c = c_ref[pl.ds(j * QB, QB), :].astype(jnp.float32)
        mx = jnp.max(jnp.where(c > 0, qk, np.float32(-1e30)), axis=1)
        sm = jnp.sum(qk * c, axis=1) * np.float32(1.0 / L_K)
        o_ref[0] = (mx - sm).reshape(rows, 128)

    M = pl.pallas_call(
        mbody,
        grid=(G, nj),
        in_specs=[pl.BlockSpec((1, QB, E), lambda i, j: (i, j, 0)),
                  pl.BlockSpec((1, L_K, E), lambda i, j: (i, 0, 0)),
                  pl.BlockSpec((L_Q, L_K), lambda i, j: (0, 0))],
        out_specs=pl.BlockSpec((1, rows, 128), lambda i, j: (i, j, 0)),
        out_shape=jax.ShapeDtypeStruct((G, L_Q // 128, 128), jnp.float32),
    )(Qh, Kh, C)

    def kbody(m_ref, o_ref):
        Mv = m_ref[...]
        gio = (jax.lax.broadcasted_iota(jnp.int32, Mv.shape, 1) * 128
               + jax.lax.broadcasted_iota(jnp.int32, Mv.shape, 2))
        lane = jax.lax.broadcasted_iota(jnp.int32, (G, 128), 1)
        out = jnp.full((G, 128), -1, jnp.int32)
        for t in range(u):
            mv = jnp.max(Mv, axis=(1, 2), keepdims=True)
            gi = jnp.min(jnp.where(Mv == mv, gio, jnp.int32(2**30)),
                         axis=(1, 2), keepdims=True)
            out = jnp.where(lane == t, gi[:, :, 0], out)
            Mv = jnp.where(gio == gi, np.float32(-np.inf), Mv)
        o_ref[...] = out.reshape(G, 1, 128)

    return pl.pallas_call(
        kbody,
        out_shape=jax.ShapeDtypeStruct((G, 1, 128), jnp.int32),
    )(M)


def _attn_apply(idx, Qh, Kh, Vh, u, causal):
    G, L_Q, E = Qh.shape
    L_K = Kh.shape[1]
    UP = 64
    CH = 512

    def body(idx_ref, q_ref, k_ref, v_ref, o_ref):
        V = v_ref[0]
        idp = idx_ref[0][:, :UP]
        rio = jax.lax.broadcasted_iota(jnp.int32, (L_Q, UP), 0)
        ot = jnp.where(rio == idp, np.float32(1.0), np.float32(0.0))
        qr = jax.lax.dot_general(ot, q_ref[0], (((0,), (0,)), ((), ())),
                                 precision=_PREC,
                                 preferred_element_type=jnp.float32)
        scores = jax.lax.dot_general(qr, k_ref[0], (((1,), (1,)), ((), ())),
                                     precision=_PREC,
                                     preferred_element_type=jnp.float32)
        scores = scores * np.float32(1.0 / np.sqrt(E))
        if causal:
            pcol = jnp.transpose(idp, (1, 0))
            kio = jax.lax.broadcasted_iota(jnp.int32, (UP, L_K), 1)
            scores = jnp.where(kio > pcol, np.float32(-1e30), scores)
        smax = jnp.max(scores, axis=-1, keepdims=True)
        ex = jnp.exp(scores - smax)
        attn = ex / jnp.sum(ex, axis=-1, keepdims=True)
        upd = jax.lax.dot_general(attn, V, (((1,), (0,)), ((), ())),
                                  precision=_PREC,
                                  preferred_element_type=jnp.float32)
        scat = jax.lax.dot_general(ot, upd, (((1,), (0,)), ((), ())),
                                   precision=_PREC,
                                   preferred_element_type=jnp.float32)
        keep = 1.0 - jnp.sum(ot, axis=1, keepdims=True)
        if causal:
            rio2 = jax.lax.broadcasted_iota(jnp.int32, (CH, CH), 0)
            cio2 = jax.lax.broadcasted_iota(jnp.int32, (CH, CH), 1)
            tri = jnp.where(rio2 >= cio2, np.float32(1.0), np.float32(0.0))
            run = jnp.zeros((1, E), jnp.float32)
            for cix in range(L_K // CH):
                sl = slice(cix * CH, (cix + 1) * CH)
                cc = jax.lax.dot_general(tri, V[sl, :], (((1,), (0,)), ((), ())),
                                         precision=_PREC,
                                         preferred_element_type=jnp.float32)
                o_ref[0, sl, :] = (cc + run) * keep[sl, :] + scat[sl, :]
                run = run + cc[CH - 1:CH, :]
        else:
            vm = jnp.mean(V, axis=0, keepdims=True)
            o_ref[0] = jnp.broadcast_to(vm, (L_Q, E)) * keep + scat

    return pl.pallas_call(
        body,
        grid=(G,),
        in_specs=[pl.BlockSpec((1, 1, 128), lambda i: (i, 0, 0)),
                  pl.BlockSpec((1, L_Q, E), lambda i: (i, 0, 0)),
                  pl.BlockSpec((1, L_K, E), lambda i: (i, 0, 0)),
                  pl.BlockSpec((1, L_K, E), lambda i: (i, 0, 0))],
        out_specs=pl.BlockSpec((1, L_Q, E), lambda i: (i, 0, 0)),
        out_shape=jax.ShapeDtypeStruct((G, L_Q, E), jnp.float32),
    )(idx, Qh, Kh, Vh)


def _proj_heads(x2d, W, bias, n_out, L, blk=1024):
    M = x2d.shape[0]
    B = M // L
    nL = L // blk

    def body(*refs):
        x_ref, w_ref, b_ref = refs[:3]
        outs = refs[3:]
        x = x_ref[...]
        for h in range(_H):
            y = jax.lax.dot_general(x, w_ref[h], (((1,), (0,)), ((), ())),
                                    precision=_PREC,
                                    preferred_element_type=jnp.float32)
            y = y + b_ref[h]
            for o in range(n_out):
                outs[o][h] = y[:, o * _E:(o + 1) * _E]

    def omap(m):
        return (m // nL, m % nL, 0)

    outs = pl.pallas_call(
        body,
        grid=(B * nL,),
        in_specs=[pl.BlockSpec((blk, _D), lambda m: (m, 0)),
                  pl.BlockSpec((_H, _D, n_out * _E), lambda m: (0, 0, 0)),
                  pl.BlockSpec((_H, 1, n_out * _E), lambda m: (0, 0, 0))],
        out_specs=[pl.BlockSpec((_H, blk, _E), omap)] * n_out,
        out_shape=[jax.ShapeDtypeStruct((B * _H, L, _E), jnp.float32)] * n_out,
    )(x2d, W, bias)
    return outs


def _merge_heads_proj(ctx, W, bias, res2d, ln, L, blk=1024):
    G = ctx.shape[0]
    B = G // _H
    nL = L // blk
    g, bln = ln

    def body(c_ref, w_ref, b_ref, r_ref, g_ref, bl_ref, o_ref):
        acc = r_ref[...] + b_ref[...]
        for h in range(_H):
            acc = acc + jax.lax.dot_general(
                c_ref[h], w_ref[h], (((1,), (0,)), ((), ())),
                precision=_PREC, preferred_element_type=jnp.float32)
        m = jnp.mean(acc, axis=-1, keepdims=True)
        d = acc - m
        v = jnp.mean(d * d, axis=-1, keepdims=True)
        o_ref[...] = d * jax.lax.rsqrt(v + 1e-5) * g_ref[...] + bl_ref[...]

    return pl.pallas_call(
        body,
        grid=(B * nL,),
        in_specs=[pl.BlockSpec((_H, blk, _E), lambda m, _n=nL: (m // _n, m % _n, 0)),
                  pl.BlockSpec((_H, _E, _D), lambda m: (0, 0, 0)),
                  pl.BlockSpec((1, _D), lambda m: (0, 0)),
                  pl.BlockSpec((blk, _D), lambda m: (m, 0)),
                  pl.BlockSpec((1, _D), lambda m: (0, 0)),
                  pl.BlockSpec((1, _D), lambda m: (0, 0))],
        out_specs=pl.BlockSpec((blk, _D), lambda m: (m, 0)),
        out_shape=jax.ShapeDtypeStruct((B * L, _D), jnp.float32),
    )(ctx, W, bias.reshape(1, _D), res2d, g.reshape(1, _D), bln.reshape(1, _D))


def _head_w(*ws):
    return jnp.concatenate(
        [w.T.reshape(_D, _H, _E).transpose(1, 0, 2) for w in ws], axis=2)


def _head_b(*bs):
    return jnp.concatenate(
        [b.reshape(_H, 1, _E) for b in bs], axis=2)


def _ow_heads(ow):
    return ow.T.reshape(_H, _E, _D)


def _attention(pa, xq, xkv, fold, causal):
    B, L_Q, _ = xq.shape
    L_K = xkv.shape[1]
    q2 = xq.reshape(B * L_Q, _D)
    if xq is xkv:
        Qh, Kh, Vh = _proj_heads(q2, _head_w(pa['qw'], pa['kw'], pa['vw']),
                                 _head_b(pa['qb'], pa['kb'], pa['vb']), 3, L_Q)
    else:
        kv2 = xkv.reshape(B * L_K, _D)
        (Qh,) = _proj_heads(q2, _head_w(pa['qw']), _head_b(pa['qb']), 1, L_Q)
        Kh, Vh = _proj_heads(kv2, _head_w(pa['kw'], pa['vw']),
                             _head_b(pa['kb'], pa['vb']), 2, L_K)
    C = jnp.asarray(_sample_counts(fold, L_Q, L_K))
    u = _u_of(L_Q)
    idx = _topk_queries(Qh, Kh, C, u)
    return _attn_apply(idx, Qh, Kh, Vh, u, causal)


def _embed(x, x_mark, conv_w, temp_w):
    B, L, Cin = x.shape
    xcat = jnp.concatenate(
        [jnp.roll(x, 1, axis=1), x, jnp.roll(x, -1, axis=1), x_mark,
         jnp.zeros((B, L, 128 - 3 * Cin - _TF), jnp.float32)], axis=-1)
    W = jnp.concatenate(
        [conv_w[:, :, 0].T, conv_w[:, :, 1].T, conv_w[:, :, 2].T, temp_w.T,
         jnp.zeros((128 - 3 * Cin - _TF, _D), jnp.float32)], axis=0)
    pos = jnp.asarray(_pos_table(L))
    return _mm(xcat.reshape(B * L, 128), W, add=pos)


def _encoder_layer(p, x, fold):
    B, L, _ = x.shape
    x2 = x.reshape(B * L, _D)
    ctx = _attention(p['attn'], x, x, fold, causal=False)
    x1 = _merge_heads_proj(ctx, _ow_heads(p['attn']['ow']), p['attn']['ob'],
                           x2, (p['ln1g'], p['ln1b']), L)
    y = _mm(x1, p['c1w'].T, bias=p['c1b'], act="gelu")
    out = _mm(y, p['c2w'].T, bias=p['c2b'], res=x1, ln=(p['ln2g'], p['ln2b']))
    return out.reshape(B, L, _D)


def _distil(p, x):
    B, L, _ = x.shape
    xcat = jnp.concatenate(
        [jnp.roll(x, 1, axis=1), x, jnp.roll(x, -1, axis=1)], axis=-1)
    W = jnp.concatenate(
        [p['w'][:, :, 0].T, p['w'][:, :, 1].T, p['w'][:, :, 2].T], axis=0)
    y = _mm(xcat.reshape(B * L, 3 * _D), W, bias=p['b'],
            scale=1.0 / np.sqrt(1.0 + 1e-5), act="elu")
    y = y.reshape(B, L, _D)
    e = y[:, 0::2, :]
    o = y[:, 1::2, :]

    def body(e_ref, o_ref, out_ref):
        ev = e_ref[0]
        ov = o_ref[0]
        om1 = jnp.concatenate(
            [jnp.full((1, _D), -np.inf, jnp.float32), ov[:-1, :]], axis=0)
        out_ref[0] = jnp.maximum(jnp.maximum(ev, ov), om1)

    Lh = L // 2
    return pl.pallas_call(
        body,
        grid=(B,),
        in_specs=[pl.BlockSpec((1, Lh, _D), lambda i: (i, 0, 0)),
                  pl.BlockSpec((1, Lh, _D), lambda i: (i, 0, 0))],
        out_specs=pl.BlockSpec((1, Lh, _D), lambda i: (i, 0, 0)),
        out_shape=jax.ShapeDtypeStruct((B, Lh, _D), jnp.float32),
    )(e, o)


def _decoder_layer(p, x, cross, f1, f2):
    B, L, _ = x.shape
    Lc = cross.shape[1]
    x2 = x.reshape(B * L, _D)
    ctx = _attention(p['self'], x, x, f1, causal=True)
    x1 = _merge_heads_proj(ctx, _ow_heads(p['self']['ow']), p['self']['ob'],
                           x2, (p['ln1g'], p['ln1b']), L)
    ctx2 = _attention(p['cross'], x1.reshape(B, L, _D), cross, f2, causal=False)
    x2b = _merge_heads_proj(ctx2, _ow_heads(p['cross']['ow']), p['cross']['ob'],
                            x1, (p['ln2g'], p['ln2b']), L)
    y = _mm(x2b, p['c1w'].T, bias=p['c1b'], act="gelu")
    out = _mm(y, p['c2w'].T, bias=p['c2b'], res=x2b, ln=(p['ln3g'], p['ln3b']))
    return out.reshape(B, L, _D)


def kernel(x_enc, x_mark_enc, x_dec, x_mark_dec, params):
    p = params
    B, L_e, _ = x_enc.shape
    L_d = x_dec.shape[1]

    enc = _embed(x_enc, x_mark_enc, p['enc_conv_w'], p['enc_temp_w'])
    h = _encoder_layer(p['enc0'], enc.reshape(B, L_e, _D), 0)
    h = _distil(p['distil0'], h)
    Lh = h.shape[1]
    h = _encoder_layer(p['enc1'], h, 1)
    h2 = _layernorm(h.reshape(B * Lh, _D), p['enc_ng'], p['enc_nb'])
    h = h2.reshape(B, Lh, _D)

    dec = _embed(x_dec, x_mark_dec, p['dec_conv_w'], p['dec_temp_w'])
    d = _decoder_layer(p['dec0'], dec.reshape(B, L_d, _D), h, 2, 3)
    d2 = _layernorm(d.reshape(B * L_d, _D), p['dec_ng'], p['dec_nb'])

    d_last = d2.reshape(B, L_d, _D)[:, -_PRED_LEN:, :].reshape(B * _PRED_LEN, _D)
    Wp = jnp.concatenate(
        [p['proj_w'].T, jnp.zeros((_D, 128 - p['proj_w'].shape[0]), jnp.float32)],
        axis=1)
    bp = jnp.concatenate(
        [p['proj_b'], jnp.zeros((128 - p['proj_b'].shape[0],), jnp.float32)])
    out = _mm(d_last, Wp, bias=bp)
    return out[:, :p['proj_w'].shape[0]].reshape(B, _PRED_LEN, p['proj_w'].shape[0])
